# Initial kernel scaffold; baseline (speedup 1.0000x reference)
import jax
import jax.numpy as jnp
from jax.experimental import pallas as pl


def kernel(heads, relations, tails, entity_emb, relation_emb):
    raise NotImplementedError("write your pallas kernel here")


# speedup vs baseline: 1.2058x; 1.2058x over previous
import functools

import jax
import jax.numpy as jnp
from jax import lax
from jax.experimental import pallas as pl
from jax.experimental.pallas import tpu as pltpu
from jax.experimental.pallas import tpu_sc as plsc

NUM_ENTITIES = 100000
NUM_RELATIONS = 1000
EMBED_DIM = 64
BATCH = 16384

NC = 2
NS = 16
L = 16
NW = NC * NS
B_PER_W = BATCH // NW
CHUNKS = EMBED_DIM // L


def _vsqrt(x):
    i = plsc.bitcast(x, jnp.int32)
    y = plsc.bitcast(jnp.int32(0x5F3759DF) - (i >> 1), jnp.float32)
    for _ in range(3):
        y = y * (1.5 - 0.5 * x * y * y)
    return x * y


def _body(heads_hbm, relations_hbm, tails_hbm, ent_hbm, rel_hbm, out_hbm,
          idx_h, idx_r, idx_t, h_rows, r_rows, t_rows, out_v, sem):
    wid = lax.axis_index("s") * NC + lax.axis_index("c")
    base = wid * B_PER_W

    pltpu.sync_copy(heads_hbm.at[pl.ds(base, B_PER_W)], idx_h)
    pltpu.sync_copy(relations_hbm.at[pl.ds(base, B_PER_W)], idx_r)
    pltpu.sync_copy(tails_hbm.at[pl.ds(base, B_PER_W)], idx_t)

    c1 = pltpu.async_copy(ent_hbm.at[idx_h], h_rows, sem)
    c2 = pltpu.async_copy(rel_hbm.at[idx_r], r_rows, sem)
    c3 = pltpu.async_copy(ent_hbm.at[idx_t], t_rows, sem)
    c1.wait()
    c2.wait()
    c3.wait()

    iota = lax.iota(jnp.int32, L)

    def group(g, _):
        base_i = g * L
        gv = jnp.zeros((L,), jnp.float32)
        for j in range(L):
            i = base_i + j
            acc = jnp.zeros((L,), jnp.float32)
            for c in range(CHUNKS):
                h = h_rows[i, pl.ds(c * L, L)]
                r = r_rows[i, pl.ds(c * L, L)]
                t = t_rows[i, pl.ds(c * L, L)]
                df = h + r - t + 1e-6
                acc = acc + df * df
            gv = jnp.where(iota == j, jnp.sum(acc), gv)
        out_v[pl.ds(base_i, L)] = _vsqrt(gv)
        return 0

    lax.fori_loop(0, B_PER_W // L, group, 0)

    pltpu.sync_copy(out_v, out_hbm.at[pl.ds(base, B_PER_W)])


@jax.jit
def _transe(heads, relations, tails, entity_emb, relation_emb):
    mesh = plsc.VectorSubcoreMesh(
        core_axis_name="c", subcore_axis_name="s", num_cores=NC,
        num_subcores=NS)
    return pl.kernel(
        _body,
        out_type=jax.ShapeDtypeStruct((BATCH,), jnp.float32),
        mesh=mesh,
        scratch_types=[
            pltpu.VMEM((B_PER_W,), jnp.int32),
            pltpu.VMEM((B_PER_W,), jnp.int32),
            pltpu.VMEM((B_PER_W,), jnp.int32),
            pltpu.VMEM((B_PER_W, EMBED_DIM), jnp.float32),
            pltpu.VMEM((B_PER_W, EMBED_DIM), jnp.float32),
            pltpu.VMEM((B_PER_W, EMBED_DIM), jnp.float32),
            pltpu.VMEM((B_PER_W,), jnp.float32),
            pltpu.SemaphoreType.DMA,
        ],
        compiler_params=pltpu.CompilerParams(
            needs_layout_passes=False, use_tc_tiling_on_sc=False),
    )(heads, relations, tails, entity_emb, relation_emb)


def kernel(heads, relations, tails, entity_emb, relation_emb):
    return _transe(heads, relations, tails, entity_emb, relation_emb)
